# baseline (speedup 1.0000x reference)
import functools

import jax
import jax.numpy as jnp
import numpy as np
from jax.experimental import pallas as pl
from jax.experimental.pallas import tpu as pltpu
from jax.sharding import Mesh, PartitionSpec as P

_shard_map = getattr(jax, "shard_map", None)
if _shard_map is None:
    from jax.experimental.shard_map import shard_map as _shard_map

_BATCH = 128
_VOCAB = 100000
_BLOCK_W = 2048

_KS0 = 0
_KS1 = 12345
_KS2 = 0x1BD11BDA ^ _KS0 ^ _KS1
_ROTS = (13, 15, 26, 6, 17, 29, 16, 24)


def _i32(v):
    v = v & 0xFFFFFFFF
    return jnp.int32(v - 0x100000000 if v >= 0x80000000 else v)


def _srl(x, d):
    return jax.lax.shift_right_logical(x, jnp.int32(d))


def _rotl(x, d):
    return jax.lax.shift_left(x, jnp.int32(d)) | _srl(x, 32 - d)


def _threefry_bits(flat_idx):
    ks = (_i32(_KS0), _i32(_KS1), _i32(_KS2))
    ksv = (_KS0, _KS1, _KS2)
    x1 = flat_idx + ks[1]
    x0 = x1
    first = True
    for i in range(5):
        rset = _ROTS[0:4] if i % 2 == 0 else _ROTS[4:8]
        for r in rset:
            if first:
                first = False
            else:
                x0 = x0 + x1
            x1 = _rotl(x1, r)
            x1 = x0 ^ x1
        x0 = x0 + ks[(i + 1) % 3]
        x1 = x1 + _i32(ksv[(i + 2) % 3] + i + 1)
    return x0 ^ x1


def _gumbel(flat_idx):
    bits = _threefry_bits(flat_idx)
    ubits = jax.lax.bitcast_convert_type(
        _srl(bits, 9) | _i32(0x3F800000), jnp.float32)
    expo = -jnp.log(2.0 - ubits)
    return -jnp.log(jnp.maximum(expo, 1e-10))


def _sample_kernel(shard_w, t_ref, gs_ref, sb_ref, l_ref, val_ref, idx_ref):
    j = pl.program_id(0)
    shape = l_ref.shape
    col = jax.lax.broadcasted_iota(jnp.int32, shape, 1) + j * _BLOCK_W
    row = jax.lax.broadcasted_iota(jnp.int32, shape, 0)
    gcol = col + sb_ref[0]
    g = _gumbel(row * _VOCAB + gcol)
    score = l_ref[...] / t_ref[...] + g * gs_ref[...]
    score = jnp.where(col < shard_w, score, -jnp.inf)
    m = jnp.max(score, axis=1, keepdims=True)
    idx = jnp.min(jnp.where(score == m, gcol, jnp.int32(2**30)),
                  axis=1, keepdims=True)

    @pl.when(j == 0)
    def _():
        val_ref[...] = m
        idx_ref[...] = idx

    @pl.when(j > 0)
    def _():
        upd = m > val_ref[...]
        val_ref[...] = jnp.where(upd, m, val_ref[...])
        idx_ref[...] = jnp.where(upd, idx, idx_ref[...])


def _shard_body(t_safe, g_scale, logits):
    b, w = logits.shape
    shard_base = (jax.lax.axis_index('x') * w).astype(jnp.int32)[None]
    vals, idxs = pl.pallas_call(
        functools.partial(_sample_kernel, w),
        grid=(pl.cdiv(w, _BLOCK_W),),
        in_specs=[
            pl.BlockSpec((b, 1), lambda j: (0, 0)),
            pl.BlockSpec((b, 1), lambda j: (0, 0)),
            pl.BlockSpec(memory_space=pltpu.SMEM),
            pl.BlockSpec((b, _BLOCK_W), lambda j: (0, j)),
        ],
        out_specs=[
            pl.BlockSpec((b, 1), lambda j: (0, 0)),
            pl.BlockSpec((b, 1), lambda j: (0, 0)),
        ],
        out_shape=[
            jax.ShapeDtypeStruct((b, 1), jnp.float32),
            jax.ShapeDtypeStruct((b, 1), jnp.int32),
        ],
        compiler_params=pltpu.CompilerParams(
            dimension_semantics=("arbitrary",)),
    )(t_safe[:, None], g_scale[:, None], shard_base, logits)
    return vals.reshape(1, b), idxs.reshape(1, b)


def kernel(logits, temperatures):
    b, v = logits.shape
    greedy = temperatures <= 1e-10
    t_safe = jnp.where(greedy, 1.0, temperatures).astype(jnp.float32)
    g_scale = jnp.where(greedy, 0.0, 1.0).astype(jnp.float32)
    devs = jax.devices()[:2]
    mesh = Mesh(np.array(devs), ('x',))
    vals, idxs = _shard_map(
        _shard_body,
        mesh=mesh,
        in_specs=(P(), P(), P(None, 'x')),
        out_specs=(P('x', None), P('x', None)),
        check_vma=False,
    )(t_safe, g_scale, logits.astype(jnp.float32))
    best = jnp.argmax(jnp.where(
        vals == jnp.max(vals, axis=0, keepdims=True),
        -jnp.arange(vals.shape[0], dtype=jnp.float32)[:, None], -jnp.inf),
        axis=0)
    return jnp.take_along_axis(idxs, best[None], axis=0).reshape(b)


# speedup vs baseline: 1.7961x; 1.7961x over previous
import jax
import jax.numpy as jnp
from jax.experimental import pallas as pl
from jax.experimental.pallas import tpu as pltpu

_BATCH = 128
_VOCAB = 100000
_BLOCK_W = 4096

_KS0 = 0
_KS1 = 12345
_KS2 = 0x1BD11BDA ^ _KS0 ^ _KS1
_ROTS = (13, 15, 26, 6, 17, 29, 16, 24)


def _i32(v):
    v = v & 0xFFFFFFFF
    return jnp.int32(v - 0x100000000 if v >= 0x80000000 else v)


def _srl(x, d):
    return jax.lax.shift_right_logical(x, jnp.int32(d))


def _rotl(x, d):
    return jax.lax.shift_left(x, jnp.int32(d)) | _srl(x, 32 - d)


def _threefry_bits(flat_idx):
    ks = (_i32(_KS0), _i32(_KS1), _i32(_KS2))
    ksv = (_KS0, _KS1, _KS2)
    x1 = flat_idx + ks[1]
    x0 = x1
    first = True
    for i in range(5):
        rset = _ROTS[0:4] if i % 2 == 0 else _ROTS[4:8]
        for r in rset:
            if first:
                first = False
            else:
                x0 = x0 + x1
            x1 = _rotl(x1, r)
            x1 = x0 ^ x1
        x0 = x0 + ks[(i + 1) % 3]
        x1 = x1 + _i32(ksv[(i + 2) % 3] + i + 1)
    return x0 ^ x1


def _gumbel(flat_idx):
    bits = _threefry_bits(flat_idx)
    ubits = jax.lax.bitcast_convert_type(
        _srl(bits, 9) | _i32(0x3F800000), jnp.float32)
    expo = -jnp.log(2.0 - ubits)
    return -jnp.log(jnp.maximum(expo, 1e-10))


def _sample_kernel(t_ref, l_ref, out_ref, best_ref, inv_t_ref, gs_ref,
                   rowbase_ref):
    j = pl.program_id(0)

    @pl.when(j == 0)
    def _():
        t = t_ref[...]
        greedy = t <= 1e-10
        inv_t_ref[...] = 1.0 / jnp.where(greedy, 1.0, t)
        gs_ref[...] = jnp.where(greedy, 0.0, 1.0)
        rowbase_ref[...] = (
            jax.lax.broadcasted_iota(jnp.int32, t.shape, 0) * _VOCAB)

    shape = l_ref.shape
    col = jax.lax.broadcasted_iota(jnp.int32, shape, 1) + j * _BLOCK_W
    g = _gumbel(rowbase_ref[...] + col)
    score = l_ref[...] * inv_t_ref[...] + g * gs_ref[...]
    score = jnp.where(col < _VOCAB, score, -jnp.inf)
    m = jnp.max(score, axis=1, keepdims=True)
    idx = jnp.min(jnp.where(score == m, col, jnp.int32(2**30)),
                  axis=1, keepdims=True)

    @pl.when(j == 0)
    def _():
        best_ref[...] = m
        out_ref[...] = idx

    @pl.when(j > 0)
    def _():
        upd = m > best_ref[...]
        best_ref[...] = jnp.where(upd, m, best_ref[...])
        out_ref[...] = jnp.where(upd, idx, out_ref[...])


def kernel(logits, temperatures):
    b, v = logits.shape
    out = pl.pallas_call(
        _sample_kernel,
        grid=(pl.cdiv(v, _BLOCK_W),),
        in_specs=[
            pl.BlockSpec((b, 1), lambda j: (0, 0)),
            pl.BlockSpec((b, _BLOCK_W), lambda j: (0, j)),
        ],
        out_specs=pl.BlockSpec((b, 1), lambda j: (0, 0)),
        out_shape=jax.ShapeDtypeStruct((b, 1), jnp.int32),
        scratch_shapes=[
            pltpu.VMEM((b, 1), jnp.float32),
            pltpu.VMEM((b, 1), jnp.float32),
            pltpu.VMEM((b, 1), jnp.float32),
            pltpu.VMEM((b, 1), jnp.int32),
        ],
        compiler_params=pltpu.CompilerParams(
            dimension_semantics=("arbitrary",)),
    )(temperatures.astype(jnp.float32)[:, None], logits.astype(jnp.float32))
    return out.reshape(b)


# speedup vs baseline: 1.9733x; 1.0987x over previous
import jax
import jax.numpy as jnp
from jax.experimental import pallas as pl
from jax.experimental.pallas import tpu as pltpu

_BATCH = 128
_VOCAB = 100000
_BLOCK_W = 2048

_KS0 = 0
_KS1 = 12345
_KS2 = 0x1BD11BDA ^ _KS0 ^ _KS1
_ROTS = (13, 15, 26, 6, 17, 29, 16, 24)


def _i32(v):
    v = v & 0xFFFFFFFF
    return jnp.int32(v - 0x100000000 if v >= 0x80000000 else v)


def _srl(x, d):
    return jax.lax.shift_right_logical(x, jnp.int32(d))


def _rotl(x, d):
    return jax.lax.shift_left(x, jnp.int32(d)) | _srl(x, 32 - d)


def _threefry_bits(flat_idx):
    ks = (_i32(_KS0), _i32(_KS1), _i32(_KS2))
    ksv = (_KS0, _KS1, _KS2)
    x1 = flat_idx + ks[1]
    x0 = x1
    first = True
    for i in range(5):
        rset = _ROTS[0:4] if i % 2 == 0 else _ROTS[4:8]
        for r in rset:
            if first:
                first = False
            else:
                x0 = x0 + x1
            x1 = _rotl(x1, r)
            x1 = x0 ^ x1
        x0 = x0 + ks[(i + 1) % 3]
        x1 = x1 + _i32(ksv[(i + 2) % 3] + i + 1)
    return x0 ^ x1


def _gumbel(flat_idx):
    bits = _threefry_bits(flat_idx)
    ubits = jax.lax.bitcast_convert_type(
        _srl(bits, 9) | _i32(0x3F800000), jnp.float32)
    expo = -jnp.log(2.0 - ubits)
    return -jnp.log(jnp.maximum(expo, 1e-10))


def _sample_kernel(t_ref, l_ref, out_ref, best_ref, inv_t_ref, gs_ref,
                   rowbase_ref):
    j = pl.program_id(0)

    @pl.when(j == 0)
    def _():
        t = t_ref[...]
        greedy = t <= 1e-10
        inv_t_ref[...] = 1.0 / jnp.where(greedy, 1.0, t)
        gs_ref[...] = jnp.where(greedy, 0.0, 1.0)
        rowbase_ref[...] = (
            jax.lax.broadcasted_iota(jnp.int32, t.shape, 0) * _VOCAB)

    shape = l_ref.shape
    col = jax.lax.broadcasted_iota(jnp.int32, shape, 1) + j * _BLOCK_W
    g = _gumbel(rowbase_ref[...] + col)
    score = l_ref[...] * inv_t_ref[...] + g * gs_ref[...]
    score = jnp.where(col < _VOCAB, score, -jnp.inf)
    m = jnp.max(score, axis=1, keepdims=True)
    idx = jnp.min(jnp.where(score == m, col, jnp.int32(2**30)),
                  axis=1, keepdims=True)

    @pl.when(j == 0)
    def _():
        best_ref[...] = m
        out_ref[...] = idx

    @pl.when(j > 0)
    def _():
        upd = m > best_ref[...]
        best_ref[...] = jnp.where(upd, m, best_ref[...])
        out_ref[...] = jnp.where(upd, idx, out_ref[...])


def kernel(logits, temperatures):
    b, v = logits.shape
    out = pl.pallas_call(
        _sample_kernel,
        grid=(pl.cdiv(v, _BLOCK_W),),
        in_specs=[
            pl.BlockSpec((b, 1), lambda j: (0, 0)),
            pl.BlockSpec((b, _BLOCK_W), lambda j: (0, j)),
        ],
        out_specs=pl.BlockSpec((b, 1), lambda j: (0, 0)),
        out_shape=jax.ShapeDtypeStruct((b, 1), jnp.int32),
        scratch_shapes=[
            pltpu.VMEM((b, 1), jnp.float32),
            pltpu.VMEM((b, 1), jnp.float32),
            pltpu.VMEM((b, 1), jnp.float32),
            pltpu.VMEM((b, 1), jnp.int32),
        ],
        compiler_params=pltpu.CompilerParams(
            dimension_semantics=("arbitrary",)),
    )(temperatures.astype(jnp.float32)[:, None], logits.astype(jnp.float32))
    return out.reshape(b)
